# baseline (speedup 1.0000x reference)
import dataclasses
import functools

import jax
import jax.numpy as jnp
from jax import lax
from jax.experimental import pallas as pl
from jax.experimental.pallas import tpu as pltpu
from jax.experimental.pallas import tpu_sc as plsc

N = 10000
E = 320000
D = 128

NC = 2
NS = 16
L = 16
NW = NC * NS

NPAD = 10240
NPT = NPAD // NW
CE = 4000
NCH = E // CE
RG = 128

BM = 1280


def _mm_body(x_ref, w_ref, b_ref, p_ref, q_ref):
    w1 = w_ref[:, :D]
    w2 = w_ref[:, D:]
    xb = x_ref[...]
    dn = (((1,), (1,)), ((), ()))
    p_ref[...] = lax.dot_general(xb, w1 - w2, dn,
                                 preferred_element_type=jnp.float32) + b_ref[...]
    q_ref[...] = lax.dot_general(xb, w2, dn, preferred_element_type=jnp.float32)


def _matmul_stage(x_pad, W, b):
    return pl.pallas_call(
        _mm_body,
        grid=(NPAD // BM,),
        in_specs=[
            pl.BlockSpec((BM, D), lambda i: (i, 0)),
            pl.BlockSpec((D, 2 * D), lambda i: (0, 0)),
            pl.BlockSpec((1, D), lambda i: (0, 0)),
        ],
        out_specs=[
            pl.BlockSpec((BM, D), lambda i: (i, 0)),
            pl.BlockSpec((BM, D), lambda i: (i, 0)),
        ],
        out_shape=[jax.ShapeDtypeStruct((NPAD, D), jnp.float32)] * 2,
    )(x_pad, W, b.reshape(1, D))


def _sc_compiler_params():
    cp = pltpu.CompilerParams()
    if "needs_layout_passes" in pltpu.CompilerParams.__dataclass_fields__:
        cp = dataclasses.replace(cp, needs_layout_passes=False)
    return cp


NEP = 64


def _sc_body(p_hbm, q_hbm, src_hbm, dst_hbm, out_hbm,
             table, dst0, src0, dst1, src1, moff0, msrc0, moff1, msrc1,
             ra0, ra1, rb, semi, sema0, sema1, semb):
    wid = lax.axis_index("s") * NC + lax.axis_index("c")
    lo = wid * NPT

    neg_inf = jnp.full((L,), -jnp.inf, dtype=jnp.float32)
    zeros_i = jnp.zeros((L,), jnp.int32)

    @pl.loop(0, NPT + 1)
    def _init_table(r):
        for c in range(D // L):
            table[r, pl.ds(c * L, L)] = neg_inf

    @pl.loop(0, CE // L)
    def _init_msrc(g):
        msrc0[pl.ds(g * L, L)] = zeros_i
        msrc1[pl.ds(g * L, L)] = zeros_i

    def idx_copies(ch, dbuf, sbuf):
        base = ch * CE
        return (pltpu.make_async_copy(dst_hbm.at[pl.ds(base, CE)], dbuf, semi),
                pltpu.make_async_copy(src_hbm.at[pl.ds(base, CE)], sbuf, semi))

    def start_idx(ch, dbuf, sbuf):
        for cp in idx_copies(ch, dbuf, sbuf):
            cp.start()

    def wait_idx(ch, dbuf, sbuf):
        for cp in idx_copies(ch, dbuf, sbuf):
            cp.wait()

    def gather_copy(msrc, b, rbuf, sem):
        return pltpu.make_async_copy(q_hbm.at[msrc.at[pl.ds(b * RG, RG)]],
                                     rbuf, sem)

    def run_filter(dbuf, sbuf, moff, msrc):
        def _filter(g, cnt):
            dv = dbuf[pl.ds(g * L, L)]
            sv = sbuf[pl.ds(g * L, L)]
            m = (dv >= lo) & (dv < lo + NPT)
            plsc.store_compressed(msrc.at[pl.ds(cnt, L)], sv, mask=m)
            plsc.store_compressed(moff.at[pl.ds(cnt, L)], dv - lo, mask=m)
            nm = plsc.all_reduce_population_count(m)
            return cnt + nm[0]

        cnt = lax.fori_loop(0, CE // L, _filter, jnp.int32(0))

        padend = (cnt + (L - 1)) & ~(L - 1)
        iot = lax.iota(jnp.int32, L)
        plsc.store_scatter(moff, [cnt + iot],
                           jnp.full((L,), NPT, jnp.int32),
                           mask=(cnt + iot) < padend)
        return cnt

    def do_batch(gbase, nvalid, moff, rbuf):
        ng = (nvalid + (L - 1)) >> 4

        def _update(g2, c2):
            dlocv = moff[pl.ds(gbase + g2 * L, L)]
            for k in range(L):
                dloc = dlocv[k]
                j = g2 * L + k
                for c in range(D // L):
                    sl = pl.ds(c * L, L)
                    table[dloc, sl] = jnp.maximum(table[dloc, sl],
                                                  rbuf[j, sl])
            return c2

        lax.fori_loop(0, ng, _update, jnp.int32(0))

    def run_update(cnt, moff, msrc, ra, sema):
        gather_copy(msrc, 0, ra, sema).wait()
        nb = (cnt + (RG - 1)) >> 7

        @pl.when(nb > 1)
        def _():
            gather_copy(msrc, 1, rb, semb).start()

        do_batch(0, jnp.minimum(cnt, RG), moff, ra)

        def _extra(b, c2):
            gather_copy(msrc, b, rb, semb).wait()
            do_batch(b * RG, jnp.minimum(cnt - b * RG, RG), moff, rb)

            @pl.when(b + 1 < nb)
            def _():
                gather_copy(msrc, b + 1, rb, semb).start()

            return c2

        lax.fori_loop(1, nb, _extra, jnp.int32(0))

    start_idx(0, dst0, src0)

    def _pair(t, cnt_prev):
        ch = 2 * t
        wait_idx(ch, dst0, src0)
        start_idx(ch + 1, dst1, src1)
        cnt_a = run_filter(dst0, src0, moff0, msrc0)
        gather_copy(msrc0, 0, ra0, sema0).start()

        @pl.when(t > 0)
        def _():
            run_update(cnt_prev, moff1, msrc1, ra1, sema1)

        wait_idx(ch + 1, dst1, src1)

        @pl.when(ch + 2 < NCH)
        def _():
            start_idx(ch + 2, dst0, src0)

        cnt_b = run_filter(dst1, src1, moff1, msrc1)
        gather_copy(msrc1, 0, ra1, sema1).start()
        run_update(cnt_a, moff0, msrc0, ra0, sema0)
        return cnt_b

    cnt_last = lax.fori_loop(0, NCH // 2, _pair, jnp.int32(0))
    run_update(cnt_last, moff1, msrc1, ra1, sema1)

    @pl.loop(0, NPT // NEP)
    def _finish(s):
        pltpu.sync_copy(p_hbm.at[pl.ds(lo + s * NEP, NEP)],
                        rb.at[pl.ds(0, NEP)])

        @pl.loop(0, NEP)
        def _finish_row(r):
            for c in range(D // L):
                sl = pl.ds(c * L, L)
                table[s * NEP + r, sl] = jnp.maximum(
                    table[s * NEP + r, sl] + rb[r, sl], 0.0)

    pltpu.sync_copy(table.at[pl.ds(0, NPT)], out_hbm.at[pl.ds(lo, NPT)])


_segment_max_stage = functools.partial(
    pl.kernel,
    out_type=jax.ShapeDtypeStruct((NPAD, D), jnp.float32),
    mesh=plsc.VectorSubcoreMesh(core_axis_name="c", subcore_axis_name="s"),
    scratch_types=[
        pltpu.VMEM((NPT + 1, D), jnp.float32),
        pltpu.VMEM((CE,), jnp.int32),
        pltpu.VMEM((CE,), jnp.int32),
        pltpu.VMEM((CE,), jnp.int32),
        pltpu.VMEM((CE,), jnp.int32),
        pltpu.VMEM((CE,), jnp.int32),
        pltpu.VMEM((CE,), jnp.int32),
        pltpu.VMEM((CE,), jnp.int32),
        pltpu.VMEM((CE,), jnp.int32),
        pltpu.VMEM((RG, D), jnp.float32),
        pltpu.VMEM((RG, D), jnp.float32),
        pltpu.VMEM((RG, D), jnp.float32),
        pltpu.SemaphoreType.DMA,
        pltpu.SemaphoreType.DMA,
        pltpu.SemaphoreType.DMA,
        pltpu.SemaphoreType.DMA,
    ],
    compiler_params=_sc_compiler_params(),
)(_sc_body)


def kernel(x, edge_index, W, b):
    x_pad = jnp.pad(x, ((0, NPAD - N), (0, 0)))
    P, Q = _matmul_stage(x_pad, W, b)
    out = _segment_max_stage(P, Q, edge_index[0], edge_index[1])
    return out[:N]


# speedup vs baseline: 5.6416x; 5.6416x over previous
import dataclasses
import functools

import jax
import jax.numpy as jnp
from jax import lax
from jax.experimental import pallas as pl
from jax.experimental.pallas import tpu as pltpu
from jax.experimental.pallas import tpu_sc as plsc

N = 10000
E = 320000
D = 128

NC = 2
NS = 16
L = 16
NW = NC * NS

NPAD = 10240
NPT = NPAD // NW
CE = 16000
NCH = E // CE
RG = 128
SB = 14
NEP = 64

BM = 1280
ER = E // D


def _mm_body(x_ref, w_ref, b_ref, p_ref, q_ref):
    w1 = w_ref[:, :D]
    w2 = w_ref[:, D:]
    xb = x_ref[...]
    dn = (((1,), (1,)), ((), ()))
    p_ref[...] = lax.dot_general(xb, w1 - w2, dn,
                                 preferred_element_type=jnp.float32) + b_ref[...]
    q_ref[...] = lax.dot_general(xb, w2, dn, preferred_element_type=jnp.float32)


def _matmul_stage(x_pad, W, b):
    grid = NPAD // BM
    return pl.pallas_call(
        _mm_body,
        grid=(grid,),
        in_specs=[
            pl.BlockSpec((BM, D), lambda i: (i, 0)),
            pl.BlockSpec((D, 2 * D), lambda i: (0, 0)),
            pl.BlockSpec((1, D), lambda i: (0, 0)),
        ],
        out_specs=[
            pl.BlockSpec((BM, D), lambda i: (i, 0)),
            pl.BlockSpec((BM, D), lambda i: (i, 0)),
        ],
        out_shape=[
            jax.ShapeDtypeStruct((NPAD, D), jnp.float32),
            jax.ShapeDtypeStruct((NPAD, D), jnp.float32),
        ],
    )(x_pad, W, b.reshape(1, D))


def _pack_body(s_ref, d_ref, pk_ref):
    pk_ref[...] = (d_ref[...] << SB) | s_ref[...]


def _pack_stage(edge_index):
    return pl.pallas_call(
        _pack_body,
        out_shape=jax.ShapeDtypeStruct((ER, D), jnp.int32),
    )(edge_index[0].reshape(ER, D), edge_index[1].reshape(ER, D))


def _sc_compiler_params():
    cp = pltpu.CompilerParams()
    if "needs_layout_passes" in pltpu.CompilerParams.__dataclass_fields__:
        cp = dataclasses.replace(cp, needs_layout_passes=False)
    return cp


def _sc_body(p_hbm, q_hbm, pk_hbm, out_hbm,
             table, pk0, pk1, mpack, gidx0, doff0, gidx1, doff1,
             ra, rb, semi, sema, semb):
    wid = lax.axis_index("s") * NC + lax.axis_index("c")
    lo = wid * NPT

    neg_inf = jnp.full((L,), -jnp.inf, dtype=jnp.float32)
    smask = jnp.full((L,), (1 << SB) - 1, jnp.int32)

    @pl.loop(0, NPT + 1)
    def _init_table(r):
        for c in range(D // L):
            table[r, pl.ds(c * L, L)] = neg_inf

    def idx_copy(ch, buf):
        return pltpu.make_async_copy(pk_hbm.at[pl.ds(ch * CE, CE)], buf, semi)

    def unpack(b, gidx, doff):
        for g in range(RG // L):
            pv = mpack[pl.ds(b * RG + g * L, L)]
            gidx[pl.ds(g * L, L)] = pv & smask
            doff[pl.ds(g * L, L)] = lax.shift_right_logical(pv, SB)

    def gather_copy(gidx, rbuf, sem):
        return pltpu.make_async_copy(q_hbm.at[gidx], rbuf, sem)

    def run_filter(buf):
        lop = lo << SB

        def _filter(g, cnt):
            pv = buf[pl.ds(g * L, L)]
            dv = lax.shift_right_logical(pv, SB)
            m = (dv >= lo) & (dv < lo + NPT)
            plsc.store_compressed(mpack.at[pl.ds(cnt, L)], pv - lop, mask=m)
            nm = plsc.all_reduce_population_count(m)
            return cnt + nm[0]

        cnt = lax.fori_loop(0, CE // L, _filter, jnp.int32(0))

        padend = (cnt + (L - 1)) & ~(L - 1)
        iot = lax.iota(jnp.int32, L)
        plsc.store_scatter(mpack, [cnt + iot],
                           jnp.full((L,), NPT << SB, jnp.int32),
                           mask=(cnt + iot) < padend)
        return cnt

    def do_batch(nvalid, doff, rbuf):
        ng = (nvalid + (L - 1)) >> 4

        def _update(g2, c2):
            dlocv = doff[pl.ds(g2 * L, L)]
            for k in range(L):
                dloc = dlocv[k]
                j = g2 * L + k
                for c in range(D // L):
                    sl = pl.ds(c * L, L)
                    table[dloc, sl] = jnp.maximum(table[dloc, sl],
                                                  rbuf[j, sl])
            return c2

        lax.fori_loop(0, ng, _update, jnp.int32(0))

    def run_updates(cnt):
        nb = (cnt + (RG - 1)) >> 7

        def _pairs(u, c2):
            b0 = 2 * u
            b1 = b0 + 1

            @pl.when(b1 < nb)
            def _():
                unpack(b1, gidx1, doff1)
                gather_copy(gidx1, rb, semb).start()

            gather_copy(gidx0, ra, sema).wait()
            do_batch(jnp.minimum(cnt - b0 * RG, RG), doff0, ra)

            @pl.when(b1 < nb)
            def _():
                @pl.when(b1 + 1 < nb)
                def _():
                    unpack(b1 + 1, gidx0, doff0)
                    gather_copy(gidx0, ra, sema).start()

                gather_copy(gidx1, rb, semb).wait()
                do_batch(jnp.minimum(cnt - b1 * RG, RG), doff1, rb)

            return c2

        lax.fori_loop(0, (nb + 1) >> 1, _pairs, jnp.int32(0))

    idx_copy(0, pk0).start()

    def _chunks(t, carry):
        ch = 2 * t
        for par, (cur, nxt) in enumerate(((pk0, pk1), (pk1, pk0))):
            idx_copy(ch + par, cur).wait()

            @pl.when(ch + par + 1 < NCH)
            def _():
                idx_copy(ch + par + 1, nxt).start()

            cnt = run_filter(cur)
            unpack(0, gidx0, doff0)

            @pl.when(cnt > 0)
            def _():
                gather_copy(gidx0, ra, sema).start()
                run_updates(cnt)

        return carry

    lax.fori_loop(0, NCH // 2, _chunks, jnp.int32(0))

    @pl.loop(0, NPT // NEP)
    def _finish(s):
        pltpu.sync_copy(p_hbm.at[pl.ds(lo + s * NEP, NEP)],
                        ra.at[pl.ds(0, NEP)])

        @pl.loop(0, NEP)
        def _finish_row(r):
            for c in range(D // L):
                sl = pl.ds(c * L, L)
                table[s * NEP + r, sl] = jnp.maximum(
                    table[s * NEP + r, sl] + ra[r, sl], 0.0)

    pltpu.sync_copy(table.at[pl.ds(0, NPT)], out_hbm.at[pl.ds(lo, NPT)])


_segment_max_stage = functools.partial(
    pl.kernel,
    out_type=jax.ShapeDtypeStruct((NPAD, D), jnp.float32),
    mesh=plsc.VectorSubcoreMesh(core_axis_name="c", subcore_axis_name="s"),
    scratch_types=[
        pltpu.VMEM((NPT + 1, D), jnp.float32),
        pltpu.VMEM((CE,), jnp.int32),
        pltpu.VMEM((CE,), jnp.int32),
        pltpu.VMEM((CE,), jnp.int32),
        pltpu.VMEM((RG,), jnp.int32),
        pltpu.VMEM((RG,), jnp.int32),
        pltpu.VMEM((RG,), jnp.int32),
        pltpu.VMEM((RG,), jnp.int32),
        pltpu.VMEM((RG, D), jnp.float32),
        pltpu.VMEM((RG, D), jnp.float32),
        pltpu.SemaphoreType.DMA,
        pltpu.SemaphoreType.DMA,
        pltpu.SemaphoreType.DMA,
    ],
    compiler_params=_sc_compiler_params(),
)(_sc_body)


def kernel(x, edge_index, W, b):
    x_pad = jnp.pad(x, ((0, NPAD - N), (0, 0)))
    P, Q = _matmul_stage(x_pad, W, b)
    packed = _pack_stage(edge_index)
    out = _segment_max_stage(P, Q, packed.reshape(E))
    return out[:N]
